# baseline (device time: 109700 ns/iter reference)
import jax
import jax.numpy as jnp
from jax import lax
from jax.experimental import pallas as pl
from jax.experimental.pallas import tpu as pltpu

N_DEV = 4
HQ = 8
DH = 128
SQ = 2048
D_MODEL = 1024
WINDOW = 128
SCALE = 0.08838834764831843
CHUNK = SQ // N_DEV
QBLK = 256
KWIN = QBLK + 2 * WINDOW
HALF = D_MODEL // 2
N_HOPS = N_DEV - 1


def kernel(x, Wq, K_ext, V_ext, Wo):
    xb = x[0]
    wqb = Wq.astype(jnp.bfloat16)
    wob = Wo.astype(jnp.bfloat16)

    def body(x_ref, wq_ref, kext_ref, vext_ref, wo_ref, out_ref,
             q_ref, ctx_ref, kst_ref, vst_ref, comm_ref, sbuf_ref,
             slota_ref, slotb_ref, dma_sems, send_sems, recv_sems,
             a_send_sems, a_recv_sems, b_send_sems, b_recv_sems):
        my = lax.axis_index("i")
        left = (my + N_DEV - 1) % N_DEV
        right = (my + 1) % N_DEV

        kcp = pltpu.make_async_copy(
            kext_ref.at[0, :, pl.ds(my * HQ, HQ), :], kst_ref,
            dma_sems.at[0])
        vcp = pltpu.make_async_copy(
            vext_ref.at[0, :, pl.ds(my * HQ, HQ), :], vst_ref,
            dma_sems.at[1])
        kcp.start()
        vcp.start()

        barrier = pltpu.get_barrier_semaphore()
        for nbr in (left, right):
            pl.semaphore_signal(barrier, inc=1, device_id=(nbr,),
                                device_id_type=pl.DeviceIdType.MESH)
        pl.semaphore_wait(barrier, 2)

        q_ref[...] = (jnp.dot(x_ref[...].astype(jnp.bfloat16), wq_ref[...],
                              preferred_element_type=jnp.float32)
                      * (SCALE * 1.4426950408889634)).astype(jnp.bfloat16)
        kcp.wait()
        vcp.wait()

        def compute_chunk(c):
            q0 = c * CHUNK
            subs = []
            for sub in range(CHUNK // QBLK):
                r0 = q0 + sub * QBLK
                kw = jnp.minimum(jnp.maximum(r0 - WINDOW, 0), SQ - KWIN)
                ii = r0 + lax.broadcasted_iota(jnp.int32, (QBLK, KWIN), 0)
                jj = kw + lax.broadcasted_iota(jnp.int32, (QBLK, KWIN), 1)
                bias = jnp.where(jnp.abs(ii - jj) <= WINDOW,
                                 jnp.float32(0), jnp.float32(-1e9))
                subs.append((r0, kw, bias))
            for h in range(HQ):
                for r0, kw, bias in subs:
                    ks = kst_ref[pl.ds(kw, KWIN), h, :].astype(jnp.bfloat16)
                    vs = vst_ref[pl.ds(kw, KWIN), h, :].astype(jnp.bfloat16)
                    qs = q_ref[pl.ds(r0, QBLK), h * DH:(h + 1) * DH]
                    s = lax.dot_general(
                        qs, ks, (((1,), (1,)), ((), ())),
                        preferred_element_type=jnp.float32)
                    w = jnp.exp2(s + bias)
                    wsum = jnp.sum(w, axis=1, keepdims=True)
                    ctx = jnp.dot(w.astype(jnp.bfloat16), vs,
                                  preferred_element_type=jnp.float32) / wsum
                    ctx_ref[pl.ds(r0, QBLK), h * DH:(h + 1) * DH] = (
                        ctx.astype(jnp.bfloat16))
            out_ref[0, pl.ds(q0, CHUNK), :] = jnp.dot(
                ctx_ref[pl.ds(q0, CHUNK), :], wo_ref[...],
                preferred_element_type=jnp.float32)

        def ring_rdma(ring, hop, src):
            dev = right if ring == 0 else left
            return pltpu.make_async_remote_copy(
                src_ref=src,
                dst_ref=comm_ref.at[ring, hop],
                send_sem=send_sems.at[ring, hop],
                recv_sem=recv_sems.at[ring, hop],
                device_id=(dev,),
                device_id_type=pl.DeviceIdType.MESH)

        def stage_and_start(s_):
            cs0 = (my - s_ + N_DEV) % N_DEV
            cs1 = (my + s_) % N_DEV
            sbuf_ref[0] = out_ref[0, pl.ds(cs0 * CHUNK, CHUNK),
                                  0:HALF].astype(jnp.bfloat16)
            sbuf_ref[1] = out_ref[0, pl.ds(cs1 * CHUNK, CHUNK),
                                  HALF:D_MODEL].astype(jnp.bfloat16)
            r0 = ring_rdma(0, s_, sbuf_ref.at[0])
            r1 = ring_rdma(1, s_, sbuf_ref.at[1])
            r0.start()
            r1.start()
            return r0, r1

        def wait_and_add(s_, r0, r1):
            r0.wait()
            r1.wait()
            cr0 = (my - s_ - 1 + N_DEV) % N_DEV
            cr1 = (my + s_ + 1) % N_DEV
            out_ref[0, pl.ds(cr0 * CHUNK, CHUNK), 0:HALF] = (
                out_ref[0, pl.ds(cr0 * CHUNK, CHUNK), 0:HALF]
                + comm_ref[0, s_].astype(jnp.float32))
            out_ref[0, pl.ds(cr1 * CHUNK, CHUNK), HALF:D_MODEL] = (
                out_ref[0, pl.ds(cr1 * CHUNK, CHUNK), HALF:D_MODEL]
                + comm_ref[1, s_].astype(jnp.float32))

        compute_chunk(my)
        h0 = stage_and_start(0)
        compute_chunk((my + 1) % N_DEV)
        compute_chunk((my + N_DEV - 1) % N_DEV)
        wait_and_add(0, *h0)
        h1 = stage_and_start(1)
        compute_chunk((my + 2) % N_DEV)
        wait_and_add(1, *h1)
        h2 = stage_and_start(2)
        wait_and_add(2, *h2)

        own0 = (my + 1) % N_DEV
        own1 = (my + N_DEV - 1) % N_DEV
        sbuf_ref[0] = out_ref[0, pl.ds(own0 * CHUNK, CHUNK),
                              0:HALF].astype(jnp.bfloat16)
        sbuf_ref[1] = out_ref[0, pl.ds(own1 * CHUNK, CHUNK),
                              HALF:D_MODEL].astype(jnp.bfloat16)

        def a_rdma(direction, half):
            dev = right if direction == 0 else left
            return pltpu.make_async_remote_copy(
                src_ref=sbuf_ref.at[half],
                dst_ref=slota_ref.at[direction, half],
                send_sem=a_send_sems.at[direction, half],
                recv_sem=a_recv_sems.at[direction, half],
                device_id=(dev,),
                device_id_type=pl.DeviceIdType.MESH)

        a_sends = []
        for direction in range(2):
            for half in range(2):
                rdma = a_rdma(direction, half)
                rdma.start()
                a_sends.append(rdma)

        a_rdma(0, 0).wait_recv()
        a_rdma(1, 1).wait_recv()
        b0 = pltpu.make_async_remote_copy(
            src_ref=slota_ref.at[0, 0], dst_ref=slotb_ref.at[0],
            send_sem=b_send_sems.at[0], recv_sem=b_recv_sems.at[0],
            device_id=(right,), device_id_type=pl.DeviceIdType.MESH)
        b1 = pltpu.make_async_remote_copy(
            src_ref=slota_ref.at[1, 1], dst_ref=slotb_ref.at[1],
            send_sem=b_send_sems.at[1], recv_sem=b_recv_sems.at[1],
            device_id=(left,), device_id_type=pl.DeviceIdType.MESH)
        b0.start()
        b1.start()

        out_ref[0, pl.ds(((my) % N_DEV) * CHUNK, CHUNK), 0:HALF] = (
            slota_ref[0, 0].astype(jnp.float32))
        out_ref[0, pl.ds(((my) % N_DEV) * CHUNK, CHUNK), HALF:D_MODEL] = (
            slota_ref[1, 1].astype(jnp.float32))
        a_rdma(0, 1).wait_recv()
        out_ref[0, pl.ds(((my + 2) % N_DEV) * CHUNK, CHUNK),
                HALF:D_MODEL] = slota_ref[0, 1].astype(jnp.float32)
        a_rdma(1, 0).wait_recv()
        out_ref[0, pl.ds(((my + 2) % N_DEV) * CHUNK, CHUNK), 0:HALF] = (
            slota_ref[1, 0].astype(jnp.float32))

        b0.wait()
        b1.wait()
        out_ref[0, pl.ds(((my + N_DEV - 1) % N_DEV) * CHUNK, CHUNK),
                0:HALF] = slotb_ref[0].astype(jnp.float32)
        out_ref[0, pl.ds(((my + 1) % N_DEV) * CHUNK, CHUNK),
                HALF:D_MODEL] = slotb_ref[1].astype(jnp.float32)
        for rdma in a_sends:
            rdma.wait_send()

    out_shape = jax.ShapeDtypeStruct((1, SQ, D_MODEL), jnp.float32)
    return pl.pallas_call(
        body,
        out_shape=out_shape,
        in_specs=[
            pl.BlockSpec(memory_space=pltpu.VMEM),
            pl.BlockSpec(memory_space=pltpu.VMEM),
            pl.BlockSpec(memory_space=pl.ANY),
            pl.BlockSpec(memory_space=pl.ANY),
            pl.BlockSpec(memory_space=pltpu.VMEM),
        ],
        out_specs=pl.BlockSpec(memory_space=pltpu.VMEM),
        scratch_shapes=[
            pltpu.VMEM((SQ, HQ * DH), jnp.bfloat16),
            pltpu.VMEM((SQ, HQ * DH), jnp.bfloat16),
            pltpu.VMEM((SQ, HQ, DH), jnp.float32),
            pltpu.VMEM((SQ, HQ, DH), jnp.float32),
            pltpu.VMEM((2, N_HOPS, CHUNK, HALF), jnp.bfloat16),
            pltpu.VMEM((2, CHUNK, HALF), jnp.bfloat16),
            pltpu.VMEM((2, 2, CHUNK, HALF), jnp.bfloat16),
            pltpu.VMEM((2, CHUNK, HALF), jnp.bfloat16),
            pltpu.SemaphoreType.DMA((2,)),
            pltpu.SemaphoreType.DMA((2, N_HOPS)),
            pltpu.SemaphoreType.DMA((2, N_HOPS)),
            pltpu.SemaphoreType.DMA((2, 2)),
            pltpu.SemaphoreType.DMA((2, 2)),
            pltpu.SemaphoreType.DMA((2,)),
            pltpu.SemaphoreType.DMA((2,)),
        ],
        compiler_params=pltpu.CompilerParams(
            collective_id=0, vmem_limit_bytes=61 * 1024 * 1024),
    )(xb, wqb, K_ext, V_ext, wob)


# device time: 100662 ns/iter; 1.0898x vs baseline; 1.0898x over previous
import jax
import jax.numpy as jnp
from jax import lax
from jax.experimental import pallas as pl
from jax.experimental.pallas import tpu as pltpu

N_DEV = 4
HQ = 8
DH = 128
SQ = 2048
D_MODEL = 1024
WINDOW = 128
SCALE = 0.08838834764831843
CHUNK = SQ // N_DEV
KWIN = CHUNK + 2 * WINDOW
HALF = D_MODEL // 2
N_HOPS = N_DEV - 1


def kernel(x, Wq, K_ext, V_ext, Wo):
    xb = x[0]
    wqb = Wq.astype(jnp.bfloat16)
    wob = Wo.astype(jnp.bfloat16)

    def body(x_ref, wq_ref, kext_ref, vext_ref, wo_ref, out_ref,
             q_ref, ctx_ref, kst_ref, vst_ref, comm_ref, sbuf_ref,
             slota_ref, slotb_ref, dma_sems, send_sems, recv_sems,
             a_send_sems, a_recv_sems, b_send_sems, b_recv_sems):
        my = lax.axis_index("i")
        left = (my + N_DEV - 1) % N_DEV
        right = (my + 1) % N_DEV

        kcp = pltpu.make_async_copy(
            kext_ref.at[0, :, pl.ds(my * HQ, HQ), :], kst_ref,
            dma_sems.at[0])
        vcp = pltpu.make_async_copy(
            vext_ref.at[0, :, pl.ds(my * HQ, HQ), :], vst_ref,
            dma_sems.at[1])
        kcp.start()
        vcp.start()

        barrier = pltpu.get_barrier_semaphore()
        for nbr in (left, right):
            pl.semaphore_signal(barrier, inc=1, device_id=(nbr,),
                                device_id_type=pl.DeviceIdType.MESH)
        pl.semaphore_wait(barrier, 2)

        q_ref[...] = (jnp.dot(x_ref[...].astype(jnp.bfloat16), wq_ref[...],
                              preferred_element_type=jnp.float32)
                      * (SCALE * 1.4426950408889634)).astype(jnp.bfloat16)
        kcp.wait()
        vcp.wait()

        def compute_chunk(c):
            q0 = c * CHUNK
            kw = jnp.minimum(jnp.maximum(q0 - WINDOW, 0), SQ - KWIN)
            ii = q0 + lax.broadcasted_iota(jnp.int32, (CHUNK, KWIN), 0)
            jj = kw + lax.broadcasted_iota(jnp.int32, (CHUNK, KWIN), 1)
            bias = jnp.where(jnp.abs(ii - jj) <= WINDOW,
                             jnp.float32(0), jnp.float32(-1e9))
            for h in range(HQ):
                ks = kst_ref[pl.ds(kw, KWIN), h, :].astype(jnp.bfloat16)
                vs = vst_ref[pl.ds(kw, KWIN), h, :].astype(jnp.bfloat16)
                qs = q_ref[pl.ds(q0, CHUNK), h * DH:(h + 1) * DH]
                s = lax.dot_general(
                    qs, ks, (((1,), (1,)), ((), ())),
                    preferred_element_type=jnp.float32)
                w = jnp.exp2(s + bias)
                wsum = jnp.sum(w, axis=1, keepdims=True)
                ctx = jnp.dot(w.astype(jnp.bfloat16), vs,
                              preferred_element_type=jnp.float32) / wsum
                ctx_ref[pl.ds(q0, CHUNK), h * DH:(h + 1) * DH] = (
                    ctx.astype(jnp.bfloat16))
            out_ref[0, pl.ds(q0, CHUNK), :] = jnp.dot(
                ctx_ref[pl.ds(q0, CHUNK), :], wo_ref[...],
                preferred_element_type=jnp.float32)

        def ring_rdma(ring, hop, src):
            dev = right if ring == 0 else left
            return pltpu.make_async_remote_copy(
                src_ref=src,
                dst_ref=comm_ref.at[ring, hop],
                send_sem=send_sems.at[ring, hop],
                recv_sem=recv_sems.at[ring, hop],
                device_id=(dev,),
                device_id_type=pl.DeviceIdType.MESH)

        def stage_and_start(s_):
            cs0 = (my - s_ + N_DEV) % N_DEV
            cs1 = (my + s_) % N_DEV
            sbuf_ref[0] = out_ref[0, pl.ds(cs0 * CHUNK, CHUNK),
                                  0:HALF].astype(jnp.bfloat16)
            sbuf_ref[1] = out_ref[0, pl.ds(cs1 * CHUNK, CHUNK),
                                  HALF:D_MODEL].astype(jnp.bfloat16)
            r0 = ring_rdma(0, s_, sbuf_ref.at[0])
            r1 = ring_rdma(1, s_, sbuf_ref.at[1])
            r0.start()
            r1.start()
            return r0, r1

        def wait_and_add(s_, r0, r1):
            r0.wait()
            r1.wait()
            cr0 = (my - s_ - 1 + N_DEV) % N_DEV
            cr1 = (my + s_ + 1) % N_DEV
            out_ref[0, pl.ds(cr0 * CHUNK, CHUNK), 0:HALF] = (
                out_ref[0, pl.ds(cr0 * CHUNK, CHUNK), 0:HALF]
                + comm_ref[0, s_].astype(jnp.float32))
            out_ref[0, pl.ds(cr1 * CHUNK, CHUNK), HALF:D_MODEL] = (
                out_ref[0, pl.ds(cr1 * CHUNK, CHUNK), HALF:D_MODEL]
                + comm_ref[1, s_].astype(jnp.float32))

        compute_chunk(my)
        h0 = stage_and_start(0)
        compute_chunk((my + 1) % N_DEV)
        compute_chunk((my + N_DEV - 1) % N_DEV)
        wait_and_add(0, *h0)
        h1 = stage_and_start(1)
        compute_chunk((my + 2) % N_DEV)
        wait_and_add(1, *h1)
        h2 = stage_and_start(2)
        wait_and_add(2, *h2)

        own0 = (my + 1) % N_DEV
        own1 = (my + N_DEV - 1) % N_DEV
        sbuf_ref[0] = out_ref[0, pl.ds(own0 * CHUNK, CHUNK),
                              0:HALF].astype(jnp.bfloat16)
        sbuf_ref[1] = out_ref[0, pl.ds(own1 * CHUNK, CHUNK),
                              HALF:D_MODEL].astype(jnp.bfloat16)

        def a_rdma(direction, half):
            dev = right if direction == 0 else left
            return pltpu.make_async_remote_copy(
                src_ref=sbuf_ref.at[half],
                dst_ref=slota_ref.at[direction, half],
                send_sem=a_send_sems.at[direction, half],
                recv_sem=a_recv_sems.at[direction, half],
                device_id=(dev,),
                device_id_type=pl.DeviceIdType.MESH)

        a_sends = []
        for direction in range(2):
            for half in range(2):
                rdma = a_rdma(direction, half)
                rdma.start()
                a_sends.append(rdma)

        a_rdma(0, 0).wait_recv()
        a_rdma(1, 1).wait_recv()
        b0 = pltpu.make_async_remote_copy(
            src_ref=slota_ref.at[0, 0], dst_ref=slotb_ref.at[0],
            send_sem=b_send_sems.at[0], recv_sem=b_recv_sems.at[0],
            device_id=(right,), device_id_type=pl.DeviceIdType.MESH)
        b1 = pltpu.make_async_remote_copy(
            src_ref=slota_ref.at[1, 1], dst_ref=slotb_ref.at[1],
            send_sem=b_send_sems.at[1], recv_sem=b_recv_sems.at[1],
            device_id=(left,), device_id_type=pl.DeviceIdType.MESH)
        b0.start()
        b1.start()

        out_ref[0, pl.ds(((my) % N_DEV) * CHUNK, CHUNK), 0:HALF] = (
            slota_ref[0, 0].astype(jnp.float32))
        out_ref[0, pl.ds(((my) % N_DEV) * CHUNK, CHUNK), HALF:D_MODEL] = (
            slota_ref[1, 1].astype(jnp.float32))
        a_rdma(0, 1).wait_recv()
        out_ref[0, pl.ds(((my + 2) % N_DEV) * CHUNK, CHUNK),
                HALF:D_MODEL] = slota_ref[0, 1].astype(jnp.float32)
        a_rdma(1, 0).wait_recv()
        out_ref[0, pl.ds(((my + 2) % N_DEV) * CHUNK, CHUNK), 0:HALF] = (
            slota_ref[1, 0].astype(jnp.float32))

        b0.wait()
        b1.wait()
        out_ref[0, pl.ds(((my + N_DEV - 1) % N_DEV) * CHUNK, CHUNK),
                0:HALF] = slotb_ref[0].astype(jnp.float32)
        out_ref[0, pl.ds(((my + 1) % N_DEV) * CHUNK, CHUNK),
                HALF:D_MODEL] = slotb_ref[1].astype(jnp.float32)
        for rdma in a_sends:
            rdma.wait_send()

    out_shape = jax.ShapeDtypeStruct((1, SQ, D_MODEL), jnp.float32)
    return pl.pallas_call(
        body,
        out_shape=out_shape,
        in_specs=[
            pl.BlockSpec(memory_space=pltpu.VMEM),
            pl.BlockSpec(memory_space=pltpu.VMEM),
            pl.BlockSpec(memory_space=pl.ANY),
            pl.BlockSpec(memory_space=pl.ANY),
            pl.BlockSpec(memory_space=pltpu.VMEM),
        ],
        out_specs=pl.BlockSpec(memory_space=pltpu.VMEM),
        scratch_shapes=[
            pltpu.VMEM((SQ, HQ * DH), jnp.bfloat16),
            pltpu.VMEM((SQ, HQ * DH), jnp.bfloat16),
            pltpu.VMEM((SQ, HQ, DH), jnp.float32),
            pltpu.VMEM((SQ, HQ, DH), jnp.float32),
            pltpu.VMEM((2, N_HOPS, CHUNK, HALF), jnp.bfloat16),
            pltpu.VMEM((2, CHUNK, HALF), jnp.bfloat16),
            pltpu.VMEM((2, 2, CHUNK, HALF), jnp.bfloat16),
            pltpu.VMEM((2, CHUNK, HALF), jnp.bfloat16),
            pltpu.SemaphoreType.DMA((2,)),
            pltpu.SemaphoreType.DMA((2, N_HOPS)),
            pltpu.SemaphoreType.DMA((2, N_HOPS)),
            pltpu.SemaphoreType.DMA((2, 2)),
            pltpu.SemaphoreType.DMA((2, 2)),
            pltpu.SemaphoreType.DMA((2,)),
            pltpu.SemaphoreType.DMA((2,)),
        ],
        compiler_params=pltpu.CompilerParams(
            collective_id=0, vmem_limit_bytes=61 * 1024 * 1024),
    )(xb, wqb, K_ext, V_ext, wob)
